# baseline (speedup 1.0000x reference)
import functools

import jax
import jax.numpy as jnp
from jax import lax
from jax.experimental import pallas as pl
from jax.experimental.pallas import tpu as pltpu
from jax.experimental.pallas import tpu_sc as plsc

_B = 16384
_D = 64
_LOSS_WEIGHT = 0.1

_NC = 2
_NS = 16
_NW = _NC * _NS
_BPW = _B // _NW
_L = 16
_CH = 128
_NCH = _BPW // _CH
_NG = _BPW // _L
_QR = _D // _L
_PB = 512

_N_CENTERS = 100000
_CROWS = ((_N_CENTERS + _PB - 1) // _PB) * (_PB // 2)

_mesh = plsc.VectorSubcoreMesh(
    core_axis_name="c", subcore_axis_name="s", num_cores=_NC, num_subcores=_NS
)


@functools.partial(
    pl.kernel,
    out_type=(
        jax.ShapeDtypeStruct((_B,), jnp.float32),
        jax.ShapeDtypeStruct((_NW * _L,), jnp.float32),
    ),
    mesh=_mesh,
    compiler_params=pltpu.CompilerParams(
        needs_layout_passes=False, use_tc_tiling_on_sc=True
    ),
    scratch_types=[
        pltpu.VMEM((_NCH, _CH), jnp.int32),
        pltpu.VMEM((_NCH, _CH), jnp.int32),
        pltpu.VMEM((_BPW // 2, 2 * _D), jnp.float32),
        pltpu.VMEM((_BPW, 2 * _D), jnp.float32),
        pltpu.VMEM((_BPW,), jnp.float32),
        pltpu.VMEM((_L,), jnp.float32),
        pltpu.SemaphoreType.DMA,
        pltpu.SemaphoreType.DMA,
    ],
)
def _center_loss_sc(feat_hbm, label_hbm, centers_hbm, logits_hbm, part_hbm,
                    idx_v, idx2_v, feat_v, cent_v, logits_v, part_v,
                    fsem, gsem):
    wid = lax.axis_index("s") * _NC + lax.axis_index("c")
    base = wid * _BPW

    fcopy = pltpu.async_copy(
        feat_hbm.at[pl.ds(wid * (_BPW // 2), _BPW // 2)], feat_v, fsem
    )
    pltpu.sync_copy(label_hbm.at[pl.ds(wid * _NCH, _NCH)], idx_v)
    for j in range(_NCH):
        for t in range(_CH // _L):
            lab = idx_v[j, pl.ds(t * _L, _L)]
            idx2_v[j, pl.ds(t * _L, _L)] = (
                (lab >> 9) * (_PB // 2) + (lab & (_PB // 2 - 1))
            )
    gcopies = [
        pltpu.async_copy(
            centers_hbm.at[idx2_v.at[j]], cent_v.at[pl.ds(j * _CH, _CH)], gsem
        )
        for j in range(_NCH)
    ]
    fcopy.wait()
    for c in gcopies:
        c.wait()

    lane = lax.iota(jnp.int32, _L)

    def group_body(g, tot):
        row_sums = jnp.zeros((_L,), jnp.float32)
        lab16 = idx_v[g // (_CH // _L), pl.ds((g % (_CH // _L)) * _L, _L)]
        fhalf = g // (_NG // 2)
        for k in range(_L):
            r = g * _L + k
            coff = ((lab16[k] >> 8) & 1) * _D
            acc = jnp.zeros((_L,), jnp.float32)
            for q in range(_QR):
                f = feat_v[r - fhalf * (_BPW // 2),
                           pl.ds(fhalf * _D + q * _L, _L)]
                c = cent_v[r, pl.ds(coff + q * _L, _L)]
                diff = f - c
                acc = acc + diff * diff
            tot = tot + acc
            row_sums = jnp.where(lane == k, jnp.sum(acc), row_sums)
        logits_v[pl.ds(g * _L, _L)] = row_sums
        return tot

    tot = lax.fori_loop(0, _NG, group_body, jnp.zeros((_L,), jnp.float32))
    part_v[...] = tot

    pltpu.sync_copy(logits_v, logits_hbm.at[pl.ds(base, _BPW)])
    pltpu.sync_copy(part_v, part_hbm.at[pl.ds(wid * _L, _L)])


def _transpose_pack_tc(xt, n_rows):
    n = xt.shape[1]
    grid = (n + _PB - 1) // _PB

    def body(a_ref, b_ref, out_ref):
        out_ref[...] = jnp.concatenate(
            [a_ref[...].T, b_ref[...].T], axis=1
        )

    return pl.pallas_call(
        body,
        grid=(grid,),
        in_specs=[
            pl.BlockSpec((_D, _PB // 2), lambda i: (0, 2 * i)),
            pl.BlockSpec((_D, _PB // 2), lambda i: (0, 2 * i + 1)),
        ],
        out_specs=pl.BlockSpec((_PB // 2, 2 * _D), lambda i: (i, 0)),
        out_shape=jax.ShapeDtypeStruct(
            (grid * (_PB // 2), 2 * _D), jnp.float32
        ),
    )(xt, xt)


def kernel(feat, label, centers):
    label2d = label.reshape(_NW * _NCH, _CH)
    feat128 = _transpose_pack_tc(feat.T, _B)
    centers128 = _transpose_pack_tc(centers.T, _N_CENTERS)
    logits, parts = _center_loss_sc(feat128, label2d, centers128)
    loss = (_LOSS_WEIGHT * 0.5) * jnp.sum(parts)
    return logits, loss


# speedup vs baseline: 1.7208x; 1.7208x over previous
import functools

import jax
import jax.numpy as jnp
from jax import lax
from jax.experimental import pallas as pl
from jax.experimental.pallas import tpu as pltpu
from jax.experimental.pallas import tpu_sc as plsc

_B = 16384
_D = 64
_LOSS_WEIGHT = 0.1

_NC = 2
_NS = 16
_NW = _NC * _NS
_BPW = _B // _NW
_L = 16
_CH = 128
_NCH = _BPW // _CH
_GPC = _CH // _L
_QR = _D // _L

_mesh = plsc.VectorSubcoreMesh(
    core_axis_name="c", subcore_axis_name="s", num_cores=_NC, num_subcores=_NS
)


@functools.partial(
    pl.kernel,
    out_type=(
        jax.ShapeDtypeStruct((_B,), jnp.float32),
        jax.ShapeDtypeStruct((_NW * _L,), jnp.float32),
    ),
    mesh=_mesh,
    compiler_params=pltpu.CompilerParams(
        needs_layout_passes=False, use_tc_tiling_on_sc=False
    ),
    scratch_types=[
        pltpu.VMEM((_NCH, _CH), jnp.int32),
        pltpu.VMEM((_BPW, _D), jnp.float32),
        pltpu.VMEM((_BPW, _D), jnp.float32),
        pltpu.VMEM((_BPW,), jnp.float32),
        pltpu.VMEM((_L,), jnp.float32),
        pltpu.SemaphoreType.DMA,
        pltpu.SemaphoreType.DMA,
        pltpu.SemaphoreType.DMA,
        pltpu.SemaphoreType.DMA,
        pltpu.SemaphoreType.DMA,
    ],
)
def _center_loss_sc(feat_hbm, label_hbm, centers_hbm, logits_hbm, part_hbm,
                    idx_v, feat_v, cent_v, logits_v, part_v,
                    fsem, g0, g1, g2, g3):
    wid = lax.axis_index("s") * _NC + lax.axis_index("c")
    base = wid * _BPW

    fcopy = pltpu.async_copy(feat_hbm.at[pl.ds(base, _BPW)], feat_v, fsem)
    pltpu.sync_copy(label_hbm.at[pl.ds(wid * _NCH, _NCH)], idx_v)
    gsems = (g0, g1, g2, g3)
    gcopies = [
        pltpu.async_copy(
            centers_hbm.at[idx_v.at[j]], cent_v.at[pl.ds(j * _CH, _CH)],
            gsems[j],
        )
        for j in range(_NCH)
    ]
    fcopy.wait()

    lane = lax.iota(jnp.int32, _L)
    tot = jnp.zeros((_L,), jnp.float32)

    for j in range(_NCH):
        gcopies[j].wait()

        def chunk_body(g, tot, _j=j):
            row_sums = jnp.zeros((_L,), jnp.float32)
            for k in range(_L):
                r = (_j * _GPC + g) * _L + k
                acc = jnp.zeros((_L,), jnp.float32)
                for q in range(_QR):
                    f = feat_v[r, pl.ds(q * _L, _L)]
                    c = cent_v[r, pl.ds(q * _L, _L)]
                    diff = f - c
                    acc = acc + diff * diff
                tot = tot + acc
                row_sums = jnp.where(lane == k, jnp.sum(acc), row_sums)
            logits_v[pl.ds((_j * _GPC + g) * _L, _L)] = row_sums
            return tot

        tot = lax.fori_loop(0, _GPC, chunk_body, tot)

    part_v[...] = tot
    pltpu.sync_copy(logits_v, logits_hbm.at[pl.ds(base, _BPW)])
    pltpu.sync_copy(part_v, part_hbm.at[pl.ds(wid * _L, _L)])


def kernel(feat, label, centers):
    label2d = label.reshape(_NW * _NCH, _CH)
    logits, parts = _center_loss_sc(feat, label2d, centers)
    loss = (_LOSS_WEIGHT * 0.5) * jnp.sum(parts)
    return logits, loss


# speedup vs baseline: 1.7702x; 1.0287x over previous
import functools

import jax
import jax.numpy as jnp
from jax import lax
from jax.experimental import pallas as pl
from jax.experimental.pallas import tpu as pltpu
from jax.experimental.pallas import tpu_sc as plsc

_B = 16384
_D = 64
_LOSS_WEIGHT = 0.1

_NC = 2
_NS = 16
_NW = _NC * _NS
_BPW = _B // _NW
_L = 16
_CH = 128
_NCH = _BPW // _CH
_NG = _BPW // _L
_QR = _D // _L

_mesh = plsc.VectorSubcoreMesh(
    core_axis_name="c", subcore_axis_name="s", num_cores=_NC, num_subcores=_NS
)


@functools.partial(
    pl.kernel,
    out_type=(
        jax.ShapeDtypeStruct((_B,), jnp.float32),
        jax.ShapeDtypeStruct((_NW * _L,), jnp.float32),
    ),
    mesh=_mesh,
    compiler_params=pltpu.CompilerParams(
        needs_layout_passes=False, use_tc_tiling_on_sc=False
    ),
    scratch_types=[
        pltpu.VMEM((_NCH, _CH), jnp.int32),
        pltpu.VMEM((_BPW * _D,), jnp.float32),
        pltpu.VMEM((_BPW, _D), jnp.float32),
        pltpu.VMEM((_BPW,), jnp.float32),
        pltpu.VMEM((_L,), jnp.float32),
        pltpu.SemaphoreType.DMA,
        pltpu.SemaphoreType.DMA,
    ],
)
def _center_loss_sc(feat_hbm, label_hbm, centers_hbm, logits_hbm, part_hbm,
                    idx_v, feat_v, cent_v, logits_v, part_v, fsem, gsem):
    wid = lax.axis_index("s") * _NC + lax.axis_index("c")
    base = wid * _BPW

    pltpu.sync_copy(label_hbm.at[pl.ds(wid * _NCH, _NCH)], idx_v)
    fcopy = pltpu.async_copy(
        feat_hbm.at[pl.ds(base * _D, _BPW * _D)], feat_v, fsem
    )
    gcopies = [
        pltpu.async_copy(
            centers_hbm.at[idx_v.at[j]], cent_v.at[pl.ds(j * _CH, _CH)], gsem
        )
        for j in range(_NCH)
    ]
    fcopy.wait()
    for c in gcopies:
        c.wait()

    lane = lax.iota(jnp.int32, _L)

    def group_body(g, tot):
        row_sums = jnp.zeros((_L,), jnp.float32)
        for k in range(_L):
            r = g * _L + k
            acc = jnp.zeros((_L,), jnp.float32)
            for q in range(_QR):
                f = feat_v[pl.ds(r * _D + q * _L, _L)]
                c = cent_v[r, pl.ds(q * _L, _L)]
                diff = f - c
                acc = acc + diff * diff
            tot = tot + acc
            row_sums = jnp.where(lane == k, jnp.sum(acc), row_sums)
        logits_v[pl.ds(g * _L, _L)] = row_sums
        return tot

    tot = lax.fori_loop(0, _NG, group_body, jnp.zeros((_L,), jnp.float32))
    part_v[...] = tot

    pltpu.sync_copy(logits_v, logits_hbm.at[pl.ds(base, _BPW)])
    pltpu.sync_copy(part_v, part_hbm.at[pl.ds(wid * _L, _L)])


def kernel(feat, label, centers):
    feat_flat = feat.reshape(_B * _D)
    label2d = label.reshape(_NW * _NCH, _CH)
    logits, parts = _center_loss_sc(feat_flat, label2d, centers)
    loss = (_LOSS_WEIGHT * 0.5) * jnp.sum(parts)
    return logits, loss
